# baseline (speedup 1.0000x reference)
import functools

import jax
import jax.numpy as jnp
from jax.experimental import pallas as pl
from jax.experimental.pallas import tpu as pltpu

_ANCHOR_W = (10.0, 16.0, 33.0)
_ANCHOR_H = (13.0, 30.0, 23.0)
_STRIDE = 8.0
_C = 85
_G = 64
_POS = _G * _G
_B = 32
_CHUNK = 128
_K = _POS // _CHUNK


def _decode_body(x_ref, o_ref, scr_ref):
    k = pl.program_id(0)
    a = pl.program_id(1)

    @pl.when(a == 0)
    def _():
        scr_ref[...] = jnp.transpose(x_ref[...], (2, 0, 1))

    ta = scr_ref[pl.ds(a * _C, _C)]
    e = jnp.exp(ta)
    sig = e / (1.0 + e)

    e_h = e[0:8]
    sig_h = sig[0:8]
    c_io = jax.lax.broadcasted_iota(jnp.int32, e_h.shape, 0)
    g_io = jax.lax.broadcasted_iota(jnp.int32, e_h.shape, 2)
    gx = (g_io & (_G - 1)).astype(jnp.float32)
    gy = (k * (_CHUNK // _G) + (g_io >> 6)).astype(jnp.float32)

    aw = jnp.where(a == 0, _ANCHOR_W[0],
                   jnp.where(a == 1, _ANCHOR_W[1], _ANCHOR_W[2]))
    ah = jnp.where(a == 0, _ANCHOR_H[0],
                   jnp.where(a == 1, _ANCHOR_H[1], _ANCHOR_H[2]))

    head = jnp.where(c_io == 0, (sig_h + gx) * _STRIDE,
           jnp.where(c_io == 1, (sig_h + gy) * _STRIDE,
           jnp.where(c_io == 2, e_h * aw,
           jnp.where(c_io == 3, e_h * ah, sig_h))))
    o_ref[...] = jnp.concatenate([head, sig[8:]], axis=0)


@jax.jit
def kernel(x):
    xt = jnp.transpose(x, (0, 2, 3, 1)).reshape(_B, _POS, 3 * _C)
    y = pl.pallas_call(
        _decode_body,
        grid=(_K, 3),
        in_specs=[pl.BlockSpec((_B, _CHUNK, 3 * _C), lambda k, a: (0, k, 0))],
        out_specs=pl.BlockSpec((_C, _B, _CHUNK), lambda k, a: (0, 0, a * _K + k)),
        out_shape=jax.ShapeDtypeStruct((_C, _B, 3 * _POS), jnp.float32),
        scratch_shapes=[pltpu.VMEM((3 * _C, _B, _CHUNK), jnp.float32)],
    )(xt)
    return jnp.transpose(y, (1, 2, 0))


# speedup vs baseline: 1.1294x; 1.1294x over previous
import functools

import jax
import jax.numpy as jnp
from jax.experimental import pallas as pl
from jax.experimental.pallas import tpu as pltpu

_ANCHOR_W = (10.0, 16.0, 33.0)
_ANCHOR_H = (13.0, 30.0, 23.0)
_STRIDE = 8.0
_C = 85
_G = 64
_POS = _G * _G
_B = 32
_CHUNK = 256
_K = _POS // _CHUNK


def _decode_body(x_ref, o_ref, scr_ref):
    k = pl.program_id(0)
    a = pl.program_id(1)

    @pl.when(a == 0)
    def _():
        scr_ref[...] = jnp.transpose(x_ref[...], (2, 0, 1))

    ta = scr_ref[pl.ds(a * _C, _C)]
    e = jnp.exp(ta)
    sig = e / (1.0 + e)

    e_h = e[0:8]
    sig_h = sig[0:8]
    c_io = jax.lax.broadcasted_iota(jnp.int32, e_h.shape, 0)
    g_io = jax.lax.broadcasted_iota(jnp.int32, e_h.shape, 2)
    gx = (g_io & (_G - 1)).astype(jnp.float32)
    gy = (k * (_CHUNK // _G) + (g_io >> 6)).astype(jnp.float32)

    aw = jnp.where(a == 0, _ANCHOR_W[0],
                   jnp.where(a == 1, _ANCHOR_W[1], _ANCHOR_W[2]))
    ah = jnp.where(a == 0, _ANCHOR_H[0],
                   jnp.where(a == 1, _ANCHOR_H[1], _ANCHOR_H[2]))

    head = jnp.where(c_io == 0, (sig_h + gx) * _STRIDE,
           jnp.where(c_io == 1, (sig_h + gy) * _STRIDE,
           jnp.where(c_io == 2, e_h * aw,
           jnp.where(c_io == 3, e_h * ah, sig_h))))
    o_ref[...] = jnp.concatenate([head, sig[8:]], axis=0)


@jax.jit
def kernel(x):
    xt = jnp.transpose(x, (0, 2, 3, 1)).reshape(_B, _POS, 3 * _C)
    y = pl.pallas_call(
        _decode_body,
        grid=(_K, 3),
        in_specs=[pl.BlockSpec((_B, _CHUNK, 3 * _C), lambda k, a: (0, k, 0))],
        out_specs=pl.BlockSpec((_C, _B, _CHUNK), lambda k, a: (0, 0, a * _K + k)),
        out_shape=jax.ShapeDtypeStruct((_C, _B, 3 * _POS), jnp.float32),
        scratch_shapes=[pltpu.VMEM((3 * _C, _B, _CHUNK), jnp.float32)],
    )(xt)
    return jnp.transpose(y, (1, 2, 0))


# speedup vs baseline: 1.1316x; 1.0020x over previous
import jax
import jax.numpy as jnp
from jax.experimental import pallas as pl
from jax.experimental.pallas import tpu as pltpu

_ANCHOR_W = (10.0, 16.0, 33.0)
_ANCHOR_H = (13.0, 30.0, 23.0)
_STRIDE = 8.0
_C = 85
_G = 64
_POS = _G * _G
_B = 32
_CHUNK = 256
_K = _POS // _CHUNK


def _decode_body(x_ref, o_ref, scr_ref):
    k = pl.program_id(0)
    a = pl.program_id(1)

    @pl.when(a == 0)
    def _():
        scr_ref[...] = jnp.transpose(x_ref[...], (2, 0, 1))

    ta = scr_ref[pl.ds(a * _C, _C)]
    e = jnp.exp(ta)
    sig = e / (1.0 + e)

    e_h = e[0:8]
    sig_h = sig[0:8]
    c_io = jax.lax.broadcasted_iota(jnp.int32, e_h.shape, 0)
    g_io = jax.lax.broadcasted_iota(jnp.int32, e_h.shape, 2)
    gx = (g_io & (_G - 1)).astype(jnp.float32)
    gy = (k * (_CHUNK // _G) + (g_io >> 6)).astype(jnp.float32)

    aw = jnp.where(a == 0, _ANCHOR_W[0],
                   jnp.where(a == 1, _ANCHOR_W[1], _ANCHOR_W[2]))
    ah = jnp.where(a == 0, _ANCHOR_H[0],
                   jnp.where(a == 1, _ANCHOR_H[1], _ANCHOR_H[2]))

    head = jnp.where(c_io == 0, (sig_h + gx) * _STRIDE,
           jnp.where(c_io == 1, (sig_h + gy) * _STRIDE,
           jnp.where(c_io == 2, e_h * aw,
           jnp.where(c_io == 3, e_h * ah, sig_h))))
    o_ref[...] = jnp.concatenate([head, sig[8:]], axis=0)


@jax.jit
def kernel(x):
    xt = jnp.transpose(x, (0, 2, 3, 1)).reshape(_B, _POS, 3 * _C)
    y = pl.pallas_call(
        _decode_body,
        grid=(_K, 3),
        in_specs=[pl.BlockSpec((_B, _CHUNK, 3 * _C), lambda k, a: (0, k, 0))],
        out_specs=pl.BlockSpec((_C, _B, _CHUNK), lambda k, a: (0, 0, a * _K + k)),
        out_shape=jax.ShapeDtypeStruct((_C, _B, 3 * _POS), jnp.float32),
        scratch_shapes=[pltpu.VMEM((3 * _C, _B, _CHUNK), jnp.float32)],
    )(xt)
    return jnp.transpose(y, (1, 2, 0))
